# baseline (speedup 1.0000x reference)
import functools

import jax
import jax.numpy as jnp
from jax import lax
from jax.experimental import pallas as pl
from jax.experimental.pallas import tpu as pltpu
from jax.experimental.pallas import tpu_sc as plsc

VOCAB = 1000000
DIM = 64
HID = 128
OUT = 5
ZD = 16
PACK = 128 // ZD
S = 200
B = 4096

NC = 2
NS = 16
NW = NC * NS
BPW = B // NW
LANES = 16
NBUF = 2
G = 4
T = S // G


_FBLK = 8000


_JB = 1000
_NJ = VOCAB // PACK
_GRID = _NJ // _JB


def _fold_body(*refs):
    x_refs, w1_ref, w2_ref, o_ref = refs[:PACK], refs[PACK], refs[PACK + 1], refs[PACK + 2]
    w2 = w2_ref[...]
    w2p = jnp.concatenate(
        [w2, jnp.zeros((ZD - OUT, HID), jnp.float32)], axis=0)
    m16 = lax.dot_general(
        w2p, w1_ref[...], (((1,), (0,)), ((), ())),
        preferred_element_type=jnp.float32,
        precision=lax.Precision.HIGHEST,
    )
    for k in range(PACK):
        y = lax.dot_general(
            x_refs[k][...], m16, (((1,), (1,)), ((), ())),
            preferred_element_type=jnp.float32,
            precision=lax.Precision.HIGHEST,
        )
        o_ref[:, k * ZD:(k + 1) * ZD] = y


def _fold(table, W1, W2):
    return pl.pallas_call(
        _fold_body,
        grid=(_GRID,),
        in_specs=[
            pl.BlockSpec((_JB, DIM),
                         functools.partial(lambda k, i: (i + k * _GRID, 0), k))
            for k in range(PACK)
        ] + [
            pl.BlockSpec((HID, DIM), lambda i: (0, 0)),
            pl.BlockSpec((OUT, HID), lambda i: (0, 0)),
        ],
        out_specs=pl.BlockSpec((_JB, 128), lambda i: (i, 0)),
        out_shape=jax.ShapeDtypeStruct((_NJ, 128), jnp.float32),
    )(*([table] * PACK), W1, W2)


_mesh = plsc.VectorSubcoreMesh(core_axis_name="c", subcore_axis_name="s")


@functools.partial(
    pl.kernel,
    out_type=jax.ShapeDtypeStruct((B, ZD), jnp.float32),
    mesh=_mesh,
    scratch_types=[
        pltpu.VMEM((S, BPW), jnp.int32),
        pltpu.VMEM((NBUF, G, BPW, ZD), jnp.float32),
        pltpu.VMEM((BPW, ZD), jnp.float32),
        pltpu.SemaphoreType.DMA,
        pltpu.SemaphoreType.DMA,
    ],
    compiler_params=pltpu.CompilerParams(use_tc_tiling_on_sc=False),
)
def _pool_z(text_hbm, tz_hbm, out_hbm, idx_v, rows_v, acc_v, sem0, sem1):
    sems = (sem0, sem1)
    wid = lax.axis_index("s") * NC + lax.axis_index("c")
    base = wid * BPW

    pltpu.sync_copy(text_hbm.at[:, pl.ds(base, BPW)], idx_v)

    @plsc.parallel_loop(0, BPW, unroll=4)
    def _zero(r):
        acc_v[r, :] = jnp.zeros((ZD,), jnp.float32)

    def _issue(t, b):
        for g in range(G):
            pltpu.async_copy(
                tz_hbm.at[idx_v.at[t * G + g]], rows_v.at[b, g], sems[b]
            )

    def _wait(b):
        for g in range(G):
            pltpu.make_async_copy(
                tz_hbm.at[idx_v.at[0]], rows_v.at[b, g], sems[b]
            ).wait()

    def _accum(b):
        @plsc.parallel_loop(0, BPW, unroll=4)
        def _body(r):
            for g in range(G):
                plsc.addupdate(acc_v.at[r, :], rows_v[b, g, r, :])

    for b in range(NBUF):
        _issue(b, b)

    def body(i, carry):
        for b in range(NBUF):
            t = NBUF * i + b
            _wait(b)
            _accum(b)
            _issue(t + NBUF, b)
        return carry

    lax.fori_loop(0, T // NBUF - 1, body, 0, unroll=False)

    for b in range(NBUF):
        _wait(b)
        _accum(b)

    pltpu.sync_copy(acc_v, out_hbm.at[pl.ds(base, BPW)])


def _finish_body(zs_ref, w2_ref, b1_ref, b2_ref, o_ref):
    c = lax.dot_general(
        b1_ref[...], w2_ref[...], (((1,), (1,)), ((), ())),
        preferred_element_type=jnp.float32,
        precision=lax.Precision.HIGHEST,
    ) + b2_ref[...]
    z = zs_ref[...][:, :OUT] * (1.0 / S) + c
    z = z - jnp.max(z, axis=1, keepdims=True)
    e = jnp.exp(z)
    o_ref[...] = e / jnp.sum(e, axis=1, keepdims=True)


def _finish(zsum, W2, b1, b2):
    return pl.pallas_call(
        _finish_body,
        out_shape=jax.ShapeDtypeStruct((B, OUT), jnp.float32),
    )(zsum, W2, b1.reshape(1, HID), b2.reshape(1, OUT))


def kernel(text, table, W1, b1, W2, b2):
    packed = _fold(table, W1, W2)
    tz = packed.reshape(VOCAB, ZD)
    text2 = (text % _NJ) * PACK + text // _NJ
    zsum = _pool_z(text2, tz)
    return _finish(zsum, W2, b1, b2)


# speedup vs baseline: 1.7809x; 1.7809x over previous
import functools

import jax
import jax.numpy as jnp
from jax import lax
from jax.experimental import pallas as pl
from jax.experimental.pallas import tpu as pltpu
from jax.experimental.pallas import tpu_sc as plsc

VOCAB = 1000000
DIM = 64
HID = 128
OUT = 5
ZD = 16
PACK = 128 // ZD
S = 200
B = 4096

NC = 2
NS = 16
NW = NC * NS
BPW = B // NW
LANES = 16
NBUF = 2
G = 4
T = S // G


_FBLK = 8000


_JB = 1000
_NJ = VOCAB // PACK
_GRID = _NJ // _JB


def _fold_body(x_ref, w1_ref, w2_ref, o_ref):
    w2 = w2_ref[...]
    w2p = jnp.concatenate(
        [w2, jnp.zeros((ZD - OUT, HID), jnp.float32)], axis=0)
    m16 = lax.dot_general(
        w2p, w1_ref[...], (((1,), (0,)), ((), ())),
        preferred_element_type=jnp.float32,
        precision=lax.Precision.HIGHEST,
    )
    x16 = x_ref[...].astype(jnp.bfloat16)
    m16b = m16.astype(jnp.bfloat16)
    y = lax.dot_general(
        x16, m16b, (((1,), (1,)), ((), ())),
        preferred_element_type=jnp.float32,
    )
    for k in range(PACK):
        o_ref[:, k * ZD:(k + 1) * ZD] = y[k * _JB:(k + 1) * _JB, :]


def _fold(table, W1, W2):
    return pl.pallas_call(
        _fold_body,
        grid=(_GRID,),
        in_specs=[
            pl.BlockSpec((_FBLK, DIM), lambda i: (i, 0)),
            pl.BlockSpec((HID, DIM), lambda i: (0, 0)),
            pl.BlockSpec((OUT, HID), lambda i: (0, 0)),
        ],
        out_specs=pl.BlockSpec((_JB, 128), lambda i: (i, 0)),
        out_shape=jax.ShapeDtypeStruct((_NJ, 128), jnp.float32),
    )(table, W1, W2)


_LCH = _JB

_mesh = plsc.VectorSubcoreMesh(core_axis_name="c", subcore_axis_name="s")


@functools.partial(
    pl.kernel,
    out_type=jax.ShapeDtypeStruct((VOCAB, ZD), jnp.float32),
    mesh=_mesh,
    scratch_types=[
        pltpu.VMEM((PACK, _LCH, ZD), jnp.float32),
        pltpu.SemaphoreType.DMA,
    ],
    compiler_params=pltpu.CompilerParams(use_tc_tiling_on_sc=False),
)
def _linearize(packed_hbm, out_hbm, buf_v, sem):
    wid = lax.axis_index("s") * NC + lax.axis_index("c")
    for j in range(4):
        c = wid + NW * j

        @pl.when(c < _GRID)
        def _do():
            r0 = c * _LCH
            for k in range(PACK):
                pltpu.async_copy(
                    packed_hbm.at[pl.ds(r0, _LCH), pl.ds(k * ZD, ZD)],
                    buf_v.at[k], sem,
                )
            for k in range(PACK):
                pltpu.make_async_copy(
                    packed_hbm.at[pl.ds(0, _LCH), pl.ds(0, ZD)],
                    buf_v.at[k], sem,
                ).wait()
            for k in range(PACK):
                pltpu.async_copy(
                    buf_v.at[k],
                    out_hbm.at[pl.ds(k * _NJ + r0, _LCH)], sem,
                )
            for k in range(PACK):
                pltpu.make_async_copy(
                    buf_v.at[k],
                    out_hbm.at[pl.ds(0, _LCH)], sem,
                ).wait()


@functools.partial(
    pl.kernel,
    out_type=jax.ShapeDtypeStruct((B, ZD), jnp.float32),
    mesh=_mesh,
    scratch_types=[
        pltpu.VMEM((S, BPW), jnp.int32),
        pltpu.VMEM((NBUF, G, BPW, ZD), jnp.float32),
        pltpu.VMEM((BPW, ZD), jnp.float32),
        pltpu.SemaphoreType.DMA,
        pltpu.SemaphoreType.DMA,
    ],
    compiler_params=pltpu.CompilerParams(use_tc_tiling_on_sc=False),
)
def _pool_z(text_hbm, tz_hbm, out_hbm, idx_v, rows_v, acc_v, sem0, sem1):
    sems = (sem0, sem1)
    wid = lax.axis_index("s") * NC + lax.axis_index("c")
    base = wid * BPW

    pltpu.sync_copy(text_hbm.at[:, pl.ds(base, BPW)], idx_v)

    @plsc.parallel_loop(0, BPW, unroll=4)
    def _zero(r):
        acc_v[r, :] = jnp.zeros((ZD,), jnp.float32)

    def _issue(t, b):
        for g in range(G):
            pltpu.async_copy(
                tz_hbm.at[idx_v.at[t * G + g]], rows_v.at[b, g], sems[b]
            )

    def _wait(b):
        for g in range(G):
            pltpu.make_async_copy(
                tz_hbm.at[idx_v.at[0]], rows_v.at[b, g], sems[b]
            ).wait()

    def _accum(b):
        @plsc.parallel_loop(0, BPW, unroll=4)
        def _body(r):
            for g in range(G):
                plsc.addupdate(acc_v.at[r, :], rows_v[b, g, r, :])

    for b in range(NBUF):
        _issue(b, b)

    def body(i, carry):
        for b in range(NBUF):
            t = NBUF * i + b
            _wait(b)
            _accum(b)
            _issue(t + NBUF, b)
        return carry

    lax.fori_loop(0, T // NBUF - 1, body, 0, unroll=False)

    for b in range(NBUF):
        _wait(b)
        _accum(b)

    pltpu.sync_copy(acc_v, out_hbm.at[pl.ds(base, BPW)])


def _finish_body(zs_ref, w2_ref, b1_ref, b2_ref, o_ref):
    c = lax.dot_general(
        b1_ref[...], w2_ref[...], (((1,), (1,)), ((), ())),
        preferred_element_type=jnp.float32,
        precision=lax.Precision.HIGHEST,
    ) + b2_ref[...]
    z = zs_ref[...][:, :OUT] * (1.0 / S) + c
    z = z - jnp.max(z, axis=1, keepdims=True)
    e = jnp.exp(z)
    o_ref[...] = e / jnp.sum(e, axis=1, keepdims=True)


def _finish(zsum, W2, b1, b2):
    return pl.pallas_call(
        _finish_body,
        out_shape=jax.ShapeDtypeStruct((B, OUT), jnp.float32),
    )(zsum, W2, b1.reshape(1, HID), b2.reshape(1, OUT))


def kernel(text, table, W1, b1, W2, b2):
    packed = _fold(table, W1, W2)
    tz = _linearize(packed)
    rem = text % _FBLK
    text2 = (rem // _JB) * _NJ + (text // _FBLK) * _JB + (text % _JB)
    zsum = _pool_z(text2, tz)
    return _finish(zsum, W2, b1, b2)
